# Initial kernel scaffold; baseline (speedup 1.0000x reference)
import jax
import jax.numpy as jnp
from jax.experimental import pallas as pl


def kernel(x, special_table, molecule_table, pe):
    raise NotImplementedError("write your pallas kernel here")


# speedup vs baseline: 3.7749x; 3.7749x over previous
import functools

import jax
import jax.numpy as jnp
from jax import lax
from jax.experimental import pallas as pl
from jax.experimental.pallas import tpu as pltpu
from jax.experimental.pallas import tpu_sc as plsc

VOCAB = 100000
D = 64
B = 4096
L = 200

NC = 2
NS = 16
NW = NC * NS

ROWS = B * L
RPW = ROWS // NW
CHUNK = 100
NCHUNK = RPW // CHUNK


def _sc_kernel(fused_hbm, idx_hbm, pe_hbm, out_hbm, idx_v, pe_v, buf, gsem):
    wid = lax.axis_index("s") * NC + lax.axis_index("c")

    pltpu.sync_copy(idx_hbm.at[wid], idx_v)
    pltpu.sync_copy(pe_hbm, pe_v)

    def pair_body(cc, carry):
        for t in range(2):
            pltpu.async_copy(
                fused_hbm.at[idx_v.at[2 * cc + t]], buf.at[t], gsem
            ).wait()

        def row_body(j, carry2):
            for t in range(2):
                for g in range(4):
                    sl = pl.ds(g * 16, 16)
                    buf[t, j, sl] = buf[t, j, sl] + pe_v[t, j, sl]
            return carry2

        lax.fori_loop(0, CHUNK, row_body, 0, unroll=2)
        pltpu.sync_copy(buf, out_hbm.at[wid, cc])
        return carry

    lax.fori_loop(0, NCHUNK // 2, pair_body, 0)


@jax.jit
def kernel(x, special_table, molecule_table, pe):
    s0 = special_table[0:1]
    fused = molecule_table.at[:4].add(special_table - s0)
    pe_eff = (pe[0] + s0).reshape(2, CHUNK, D)
    idx = x.astype(jnp.int32).reshape(NW, NCHUNK, CHUNK)

    mesh = plsc.VectorSubcoreMesh(core_axis_name="c", subcore_axis_name="s")
    out = pl.kernel(
        _sc_kernel,
        out_type=jax.ShapeDtypeStruct((NW, NCHUNK // 2, 2, CHUNK, D), jnp.float32),
        mesh=mesh,
        compiler_params=pltpu.CompilerParams(use_tc_tiling_on_sc=False),
        scratch_types=[
            pltpu.VMEM((NCHUNK, CHUNK), jnp.int32),
            pltpu.VMEM((2, CHUNK, D), jnp.float32),
            pltpu.VMEM((2, CHUNK, D), jnp.float32),
            pltpu.SemaphoreType.DMA,
        ],
    )(fused, idx, pe_eff)
    return out.reshape(B, L, D)


# speedup vs baseline: 6.2760x; 1.6625x over previous
import functools

import jax
import jax.numpy as jnp
from jax import lax
from jax.experimental import pallas as pl
from jax.experimental.pallas import tpu as pltpu
from jax.experimental.pallas import tpu_sc as plsc

VOCAB = 100000
D = 64
B = 4096
L = 200

NC = 2
NS = 16
NW = NC * NS

ROWS = B * L
RPW = ROWS // NW
CHUNK = 100
NCHUNK = RPW // CHUNK


NPAIR = NCHUNK // 2
NBUF = 2


def _sc_kernel(fused_hbm, idx_hbm, pe_hbm, out_hbm, idx_v, pe_v, gbuf, obuf,
               g0, g1, o0, o1):
    wid = lax.axis_index("s") * NC + lax.axis_index("c")
    gsem = (g0, g1)
    osem = (o0, o1)

    pltpu.sync_copy(idx_hbm.at[wid], idx_v)
    pltpu.sync_copy(pe_hbm, pe_v)

    def g_copy(b, c, t):
        return pltpu.make_async_copy(
            fused_hbm.at[idx_v.at[2 * c + t]], gbuf.at[b, t], gsem[b])

    def g_start(b, c):
        for t in range(2):
            g_copy(b, c, t).start()

    def g_wait(b, c):
        for t in range(2):
            g_copy(b, c, t).wait()

    def o_copy(b, c):
        return pltpu.make_async_copy(obuf.at[b], out_hbm.at[wid, c], osem[b])

    def add_pe(b):
        @plsc.parallel_loop(0, CHUNK, unroll=4)
        def _(j):
            for t in range(2):
                for g in range(4):
                    sl = pl.ds(g * 16, 16)
                    obuf[b, t, j, sl] = gbuf[b, t, j, sl] + pe_v[t, j, sl]

    for b in range(NBUF):
        g_start(b, b)

    for b in range(NBUF):
        g_wait(b, b)
        add_pe(b)
        o_copy(b, b).start()
        g_start(b, b + NBUF)

    def steady(o, carry):
        for b in range(NBUF):
            c = o * NBUF + b
            g_wait(b, c)
            o_copy(b, c - NBUF).wait()
            add_pe(b)
            o_copy(b, c).start()
            g_start(b, c + NBUF)
        return carry

    lax.fori_loop(1, NPAIR // NBUF - 1, steady, 0)

    for b in range(NBUF):
        c = NPAIR - NBUF + b
        g_wait(b, c)
        o_copy(b, c - NBUF).wait()
        add_pe(b)
        o_copy(b, c).start()
    for b in range(NBUF):
        o_copy(b, NPAIR - NBUF + b).wait()


@jax.jit
def kernel(x, special_table, molecule_table, pe):
    s0 = special_table[0:1]
    fused = molecule_table.at[:4].add(special_table - s0)
    pe_eff = (pe[0] + s0).reshape(2, CHUNK, D)
    idx = x.astype(jnp.int32).reshape(NW, NCHUNK, CHUNK)

    mesh = plsc.VectorSubcoreMesh(core_axis_name="c", subcore_axis_name="s")
    out = pl.kernel(
        _sc_kernel,
        out_type=jax.ShapeDtypeStruct((NW, NCHUNK // 2, 2, CHUNK, D), jnp.float32),
        mesh=mesh,
        compiler_params=pltpu.CompilerParams(use_tc_tiling_on_sc=False),
        scratch_types=[
            pltpu.VMEM((NCHUNK, CHUNK), jnp.int32),
            pltpu.VMEM((2, CHUNK, D), jnp.float32),
            pltpu.VMEM((NBUF, 2, CHUNK, D), jnp.float32),
            pltpu.VMEM((NBUF, 2, CHUNK, D), jnp.float32),
            pltpu.SemaphoreType.DMA,
            pltpu.SemaphoreType.DMA,
            pltpu.SemaphoreType.DMA,
            pltpu.SemaphoreType.DMA,
        ],
    )(fused, idx, pe_eff)
    return out.reshape(B, L, D)


# speedup vs baseline: 9.8429x; 1.5683x over previous
import functools

import jax
import jax.numpy as jnp
from jax import lax
from jax.experimental import pallas as pl
from jax.experimental.pallas import tpu as pltpu
from jax.experimental.pallas import tpu_sc as plsc

VOCAB = 100000
D = 64
B = 4096
L = 200

NC = 2
NS = 16
NW = NC * NS

ROWS = B * L
RPW = ROWS // NW
CHUNK = 100
NCHUNK = RPW // CHUNK


NPAIR = NCHUNK // 2
NBUF = 2


def _sc_kernel(fused_hbm, idx_hbm, pe_hbm, out_hbm, idx_v, pe_v, gbuf, obuf,
               g0, g1, o0, o1):
    wid = lax.axis_index("s") * NC + lax.axis_index("c")
    gsem = (g0, g1)
    osem = (o0, o1)

    pltpu.sync_copy(idx_hbm.at[wid], idx_v)
    pltpu.sync_copy(pe_hbm, pe_v)

    def g_copy(b, c, t):
        return pltpu.make_async_copy(
            fused_hbm.at[idx_v.at[2 * c + t]], gbuf.at[b, t], gsem[b])

    def g_start(b, c):
        for t in range(2):
            g_copy(b, c, t).start()

    def g_wait(b, c):
        for t in range(2):
            g_copy(b, c, t).wait()

    def o_copy(b, c):
        return pltpu.make_async_copy(
            obuf.at[b], out_hbm.at[wid * NPAIR + c], osem[b])

    def add_pe(b):
        @plsc.parallel_loop(0, CHUNK // 2, unroll=2)
        def _(jp):
            for t in range(2):
                for h in range(2):
                    for g in range(4):
                        src = pl.ds(g * 16, 16)
                        dst = pl.ds(h * D + g * 16, 16)
                        obuf[b, t * 50 + jp, dst] = (
                            gbuf[b, t, 2 * jp + h, src]
                            + pe_v[t, 2 * jp + h, src])

    for b in range(NBUF):
        g_start(b, b)

    for b in range(NBUF):
        g_wait(b, b)
        add_pe(b)
        o_copy(b, b).start()
        g_start(b, b + NBUF)

    def steady(o, carry):
        for b in range(NBUF):
            c = o * NBUF + b
            g_wait(b, c)
            o_copy(b, c - NBUF).wait()
            add_pe(b)
            o_copy(b, c).start()
            g_start(b, c + NBUF)
        return carry

    lax.fori_loop(1, NPAIR // NBUF - 1, steady, 0)

    for b in range(NBUF):
        c = NPAIR - NBUF + b
        g_wait(b, c)
        o_copy(b, c - NBUF).wait()
        add_pe(b)
        o_copy(b, c).start()
    for b in range(NBUF):
        o_copy(b, NPAIR - NBUF + b).wait()


@jax.jit
def kernel(x, special_table, molecule_table, pe):
    s0 = special_table[0:1]
    fused = molecule_table.at[:4].add(special_table - s0)
    pe_eff = (pe[0] + s0).reshape(2, CHUNK, D)
    idx = x.astype(jnp.int32).reshape(NW, NCHUNK, CHUNK)

    mesh = plsc.VectorSubcoreMesh(core_axis_name="c", subcore_axis_name="s")
    out = pl.kernel(
        _sc_kernel,
        out_type=jax.ShapeDtypeStruct((B, L // 2, 2 * D), jnp.float32),
        mesh=mesh,
        compiler_params=pltpu.CompilerParams(use_tc_tiling_on_sc=False),
        scratch_types=[
            pltpu.VMEM((NCHUNK, CHUNK), jnp.int32),
            pltpu.VMEM((2, CHUNK, D), jnp.float32),
            pltpu.VMEM((NBUF, 2, CHUNK, D), jnp.float32),
            pltpu.VMEM((NBUF, L // 2, 2 * D), jnp.float32),
            pltpu.SemaphoreType.DMA,
            pltpu.SemaphoreType.DMA,
            pltpu.SemaphoreType.DMA,
            pltpu.SemaphoreType.DMA,
        ],
    )(fused, idx, pe_eff)
    return out.reshape(B, L, D)


# speedup vs baseline: 13.6778x; 1.3896x over previous
import functools

import jax
import jax.numpy as jnp
from jax import lax
from jax.experimental import pallas as pl
from jax.experimental.pallas import tpu as pltpu
from jax.experimental.pallas import tpu_sc as plsc

VOCAB = 100000
D = 64
B = 4096
L = 200

NC = 2
NS = 16
NW = NC * NS

ROWS = B * L
RPW = ROWS // NW
CHUNK = 100
NCHUNK = RPW // CHUNK


NPAIR = NCHUNK // 2
NBUF = 2


def _sc_kernel(fused_hbm, idx_hbm, pe_hbm, out_hbm, idx_v, pe_v, gbuf, obuf,
               g0, g1, o0, o1):
    wid = lax.axis_index("s") * NC + lax.axis_index("c")
    gsem = (g0, g1)
    osem = (o0, o1)

    pltpu.sync_copy(idx_hbm.at[wid], idx_v)
    pltpu.sync_copy(pe_hbm, pe_v)

    def g_copy(b, c, t):
        return pltpu.make_async_copy(
            fused_hbm.at[idx_v.at[2 * c + t]], gbuf.at[b, t], gsem[b])

    def g_start(b, c):
        for t in range(2):
            g_copy(b, c, t).start()

    def g_wait(b, c):
        for t in range(2):
            g_copy(b, c, t).wait()

    def o_copy(b, c):
        return pltpu.make_async_copy(
            obuf.at[b],
            out_hbm.at[wid * NPAIR + c, :, pl.ds(0, D)],
            osem[b])

    def add_pe(b):
        @plsc.parallel_loop(0, CHUNK, unroll=4)
        def _(j):
            for t in range(2):
                for g in range(4):
                    sl = pl.ds(g * 16, 16)
                    obuf[b, t * CHUNK + j, sl] = (
                        gbuf[b, t, j, sl] + pe_v[t, j, sl])

    for b in range(NBUF):
        g_start(b, b)

    for b in range(NBUF):
        g_wait(b, b)
        add_pe(b)
        o_copy(b, b).start()
        g_start(b, b + NBUF)

    def steady(o, carry):
        for b in range(NBUF):
            c = o * NBUF + b
            g_wait(b, c)
            o_copy(b, c - NBUF).wait()
            add_pe(b)
            o_copy(b, c).start()
            g_start(b, c + NBUF)
        return carry

    lax.fori_loop(1, NPAIR // NBUF - 1, steady, 0)

    for b in range(NBUF):
        c = NPAIR - NBUF + b
        g_wait(b, c)
        o_copy(b, c - NBUF).wait()
        add_pe(b)
        o_copy(b, c).start()
    for b in range(NBUF):
        o_copy(b, NPAIR - NBUF + b).wait()


@jax.jit
def kernel(x, special_table, molecule_table, pe):
    s0 = special_table[0:1]
    fused = molecule_table.at[:4].add(special_table - s0)
    pe_eff = (pe[0] + s0).reshape(2, CHUNK, D)
    idx = x.astype(jnp.int32).reshape(NW, NCHUNK, CHUNK)

    mesh = plsc.VectorSubcoreMesh(core_axis_name="c", subcore_axis_name="s")
    out = pl.kernel(
        _sc_kernel,
        out_type=jax.ShapeDtypeStruct((B, L, 2 * D), jnp.float32),
        mesh=mesh,
        compiler_params=pltpu.CompilerParams(use_tc_tiling_on_sc=False),
        scratch_types=[
            pltpu.VMEM((NCHUNK, CHUNK), jnp.int32),
            pltpu.VMEM((2, CHUNK, D), jnp.float32),
            pltpu.VMEM((NBUF, 2, CHUNK, D), jnp.float32),
            pltpu.VMEM((NBUF, L, D), jnp.float32),
            pltpu.SemaphoreType.DMA,
            pltpu.SemaphoreType.DMA,
            pltpu.SemaphoreType.DMA,
            pltpu.SemaphoreType.DMA,
        ],
    )(fused, idx, pe_eff)
    return out[:, :, :D]
